# baseline (speedup 1.0000x reference)
import functools

import jax
import jax.numpy as jnp
from jax import lax
from jax.experimental import pallas as pl
from jax.experimental.pallas import tpu as pltpu
from jax.experimental.pallas import tpu_sc as plsc

VOCAB = 100000
SEQ = 200
D = 128
B = 4096
LANES = 16

NC = 2
NS = 16
NW = NC * NS

B_TC = 512
B_SC = B - B_TC

SEQ_PER_W = B_SC // NW
TOK_PER_W = SEQ_PER_W * SEQ
PIECE = 40
NPIECE = SEQ // PIECE
NPAIR = SEQ_PER_W // 2
NUNIT = NPAIR * NPIECE
NSLOT = 5
LA = NSLOT - 2
BLK = 15
VPR = D // LANES

assert B_SC % (2 * NW) == 0 and NUNIT > BLK


def _sc_body(x_hbm, tok_hbm, pos_hbm, out_hbm, idx_v, pos_v, rows_v, *sems):
    gsem = sems[:NSLOT]
    osem = sems[NSLOT:]
    wid = lax.axis_index("c") * NS + lax.axis_index("s")

    pltpu.sync_copy(x_hbm.at[wid], idx_v)
    pltpu.sync_copy(pos_hbm, pos_v)

    def gathers(k, c, s):
        out = []
        for half in (0, 1):
            off = (2 * k + half) * SEQ + c * PIECE
            out.append(pltpu.make_async_copy(
                tok_hbm.at[idx_v.at[pl.ds(off, PIECE)]],
                rows_v.at[s, pl.ds(half * PIECE, PIECE)],
                gsem[s],
            ))
        return out

    def scatters(k, c, s):
        out = []
        for half in (0, 1):
            base = (wid * SEQ_PER_W + 2 * k + half) * SEQ + c * PIECE
            out.append(pltpu.make_async_copy(
                rows_v.at[s, pl.ds(half * PIECE, PIECE)],
                out_hbm.at[pl.ds(base, PIECE)],
                osem[s],
            ))
        return out

    def add_pos(c, s):
        def body(i, carry):
            for j in range(VPR):
                sl = pl.ds(j * LANES, LANES)
                p = pos_v[c * PIECE + i, sl]
                plsc.addupdate(rows_v.at[s, i, sl], p)
                plsc.addupdate(rows_v.at[s, PIECE + i, sl], p)
            return carry

        lax.fori_loop(0, PIECE, body, 0, unroll=2)

    def unit_step(u, k, b):
        c = b % NPIECE
        s = b % NSLOT
        for g in gathers(k, c, s):
            g.wait()
        add_pos(c, s)
        for sc in scatters(k, c, s):
            sc.start()
        sp = (b + LA) % NSLOT
        cp = (b + LA) % NPIECE
        c2 = (b - 2) % NPIECE
        assert sp == (b - 2) % NSLOT

        def _drain_prev():
            u2 = u - 2
            for sc in scatters(u2 // NPIECE, c2, sp):
                sc.wait()

        def _issue_next():
            up = u + LA
            for g in gathers(up // NPIECE, cp, sp):
                g.start()

        if isinstance(u, int):
            if u >= 2:
                _drain_prev()
            if u + LA < NUNIT:
                _issue_next()
        else:
            pl.when(u >= 2)(_drain_prev)
            pl.when(u + LA < NUNIT)(_issue_next)

    for u in range(LA):
        for g in gathers(0, u, u):
            g.start()

    nfull = (NUNIT // BLK) * BLK

    def outer(t, carry):
        u0 = t * BLK
        for b in range(BLK):
            u = u0 + b
            unit_step(u, u // NPIECE, b)
        return carry

    lax.fori_loop(0, NUNIT // BLK, outer, 0)

    for u in range(nfull, NUNIT):
        unit_step(u, u // NPIECE, u % BLK)
    for u in (NUNIT - 2, NUNIT - 1):
        for sc in scatters(u // NPIECE, u % NPIECE, u % NSLOT):
            sc.wait()


def _sc_embed(xw, token_table, pos):
    mesh = plsc.VectorSubcoreMesh(
        core_axis_name="c", subcore_axis_name="s", num_cores=NC, num_subcores=NS
    )
    scratch = [
        pltpu.VMEM((TOK_PER_W,), jnp.int32),
        pltpu.VMEM((SEQ, D), jnp.float32),
        pltpu.VMEM((NSLOT, 2 * PIECE, D), jnp.float32),
    ] + [pltpu.SemaphoreType.DMA] * (2 * NSLOT)
    f = pl.kernel(
        _sc_body,
        out_type=jax.ShapeDtypeStruct((B_SC * SEQ, D), jnp.float32),
        mesh=mesh,
        scratch_types=scratch,
    )
    return f(xw, token_table, pos)


def _tc_body(x_smem, table_ref, pos_ref, out_ref):
    for t8 in range(SEQ // 8):
        rows = [
            table_ref[pl.ds(x_smem[0, 0, t8 * 8 + j], 1), :] for j in range(8)
        ]
        blk = jnp.concatenate(rows, axis=0)
        out_ref[0, pl.ds(t8 * 8, 8), :] = blk + pos_ref[pl.ds(t8 * 8, 8), :]


def _tc_embed(x_tc, token_table, pos):
    return pl.pallas_call(
        _tc_body,
        grid=(B_TC,),
        in_specs=[
            pl.BlockSpec((1, 1, SEQ), lambda i: (i, 0, 0),
                         memory_space=pltpu.SMEM),
            pl.BlockSpec((VOCAB, D), lambda i: (0, 0)),
            pl.BlockSpec((SEQ, D), lambda i: (0, 0)),
        ],
        out_specs=pl.BlockSpec((1, SEQ, D), lambda i: (i, 0, 0)),
        out_shape=jax.ShapeDtypeStruct((B_TC, SEQ, D), jnp.float32),
        compiler_params=pltpu.CompilerParams(
            vmem_limit_bytes=60 * 1024 * 1024,
        ),
    )(x_tc.reshape(B_TC, 1, SEQ), token_table, pos)


@jax.jit
def _embed(x, token_table, pos):
    xw = x[:B_SC].reshape(NW, TOK_PER_W)
    sc_out = _sc_embed(xw, token_table, pos).reshape(B_SC, SEQ, D)
    tc_out = _tc_embed(x[B_SC:], token_table, pos)
    return jnp.concatenate([sc_out, tc_out], axis=0)


def kernel(x, token_table, pos_table, positions):
    pos = jnp.take(pos_table, positions, axis=0).astype(jnp.float32)
    return _embed(x.astype(jnp.int32), token_table, pos)


# speedup vs baseline: 2.0974x; 2.0974x over previous
import functools

import jax
import jax.numpy as jnp
from jax import lax
from jax.experimental import pallas as pl
from jax.experimental.pallas import tpu as pltpu
from jax.experimental.pallas import tpu_sc as plsc

VOCAB = 100000
SEQ = 200
D = 128
B = 4096
LANES = 16

NC = 2
NS = 16
NW = NC * NS
SEQ_PER_W = B // NW
TOK_PER_W = SEQ_PER_W * SEQ
PIECE = 40
NPIECE = SEQ // PIECE
NPAIR = SEQ_PER_W // 2
NUNIT = NPAIR * NPIECE
NSLOT = 5
LA = 3
BLK = 15
VPR = D // LANES


def _sc_body(x_hbm, tok_hbm, pos_hbm, out_hbm, idx_v, pos_v, rows_v, *sems):
    gsem = sems[:NSLOT]
    osem = sems[NSLOT:]
    wid = lax.axis_index("c") * NS + lax.axis_index("s")

    pltpu.sync_copy(x_hbm.at[wid], idx_v)
    pltpu.sync_copy(pos_hbm, pos_v)

    def gathers(k, c, s):
        out = []
        for half in (0, 1):
            off = (2 * k + half) * SEQ + c * PIECE
            out.append(pltpu.make_async_copy(
                tok_hbm.at[idx_v.at[pl.ds(off, PIECE)]],
                rows_v.at[s, pl.ds(half * PIECE, PIECE)],
                gsem[s],
            ))
        return out

    def scatters(k, c, s):
        out = []
        for half in (0, 1):
            base = (wid * SEQ_PER_W + 2 * k + half) * SEQ + c * PIECE
            out.append(pltpu.make_async_copy(
                rows_v.at[s, pl.ds(half * PIECE, PIECE)],
                out_hbm.at[pl.ds(base, PIECE)],
                osem[s],
            ))
        return out

    def add_pos(c, s):
        def body(i, carry):
            for j in range(VPR):
                sl = pl.ds(j * LANES, LANES)
                p = pos_v[c * PIECE + i, sl]
                plsc.addupdate(rows_v.at[s, i, sl], p)
                plsc.addupdate(rows_v.at[s, PIECE + i, sl], p)
            return carry

        lax.fori_loop(0, PIECE, body, 0, unroll=2)

    def unit_step(u, k, b):
        c = b % NPIECE
        s = b % NSLOT
        for g in gathers(k, c, s):
            g.wait()
        add_pos(c, s)
        for sc in scatters(k, c, s):
            sc.start()
        sp = (b + LA) % NSLOT
        cp = (b + LA) % NPIECE
        c2 = (b - 2) % NPIECE
        assert cp == c2 and sp == (b - 2) % NSLOT

        def _drain_prev():
            u2 = u - 2
            for sc in scatters(u2 // NPIECE, c2, sp):
                sc.wait()

        def _issue_next():
            up = u + LA
            for g in gathers(up // NPIECE, cp, sp):
                g.start()

        if isinstance(u, int):
            if u >= 2:
                _drain_prev()
            if u + LA < NUNIT:
                _issue_next()
        else:
            pl.when(u >= 2)(_drain_prev)
            pl.when(u + LA < NUNIT)(_issue_next)

    for u in range(LA):
        for g in gathers(0, u, u):
            g.start()

    nfull = (NUNIT // BLK) * BLK

    def outer(t, carry):
        u0 = t * BLK
        for b in range(BLK):
            u = u0 + b
            unit_step(u, u // NPIECE, b)
        return carry

    lax.fori_loop(0, NUNIT // BLK, outer, 0)

    for u in range(nfull, NUNIT):
        unit_step(u, u // NPIECE, u % BLK)
    for u in (NUNIT - 2, NUNIT - 1):
        for sc in scatters(u // NPIECE, u % NPIECE, u % NSLOT):
            sc.wait()


@jax.jit
def _sc_embed(xw, token_table, pos):
    mesh = plsc.VectorSubcoreMesh(
        core_axis_name="c", subcore_axis_name="s", num_cores=NC, num_subcores=NS
    )
    scratch = [
        pltpu.VMEM((TOK_PER_W,), jnp.int32),
        pltpu.VMEM((SEQ, D), jnp.float32),
        pltpu.VMEM((NSLOT, 2 * PIECE, D), jnp.float32),
    ] + [pltpu.SemaphoreType.DMA] * (2 * NSLOT)
    f = pl.kernel(
        _sc_body,
        out_type=jax.ShapeDtypeStruct((B * SEQ, D), jnp.float32),
        mesh=mesh,
        scratch_types=scratch,
    )
    return f(xw, token_table, pos)


def kernel(x, token_table, pos_table, positions):
    pos = jnp.take(pos_table, positions, axis=0).astype(jnp.float32)
    xw = x.astype(jnp.int32).reshape(NW, TOK_PER_W)
    out = _sc_embed(xw, token_table, pos)
    return out.reshape(B, SEQ, D)
